# baseline (speedup 1.0000x reference)
import functools

import jax
import jax.numpy as jnp
from jax import lax
from jax.experimental import pallas as pl
from jax.experimental.pallas import tpu as pltpu
from jax.experimental.pallas import tpu_sc as plsc

B = 4096
L = 200
D = 64
LANES = 16
NVR = D // LANES
NFULL = L // LANES
LREM = L - NFULL * LANES
NBUF = 4

_info = plsc.get_sparse_core_info()
_NC, _NS = _info.num_cores, _info.num_subcores
NW = _NC * _NS
RPW = B // NW


def _tec_body(idx_hbm, table_hbm, out_hbm, idx_all,
              rows0, rows1, rows2, rows3, t0_v, out_blk,
              sem0, sem1, sem2, sem3):
    bufs = (rows0, rows1, rows2, rows3)
    sems = (sem0, sem1, sem2, sem3)
    wid = lax.axis_index("s") * _NC + lax.axis_index("c")
    row0 = wid * RPW

    pltpu.sync_copy(idx_hbm.at[pl.ds(row0 * L, RPW * L)],
                    idx_all.at[pl.ds(0, RPW * L)])
    pltpu.sync_copy(table_hbm.at[0], t0_v)

    lane = lax.iota(jnp.int32, LANES)
    last_mask = lane < LREM
    zero = jnp.zeros((LANES,), jnp.float32)
    t0 = [t0_v[pl.ds(k * LANES, LANES)] for k in range(NVR)]

    def gather_copies(r, buf, sem):
        off = r * L
        c1 = pltpu.make_async_copy(
            table_hbm.at[idx_all.at[pl.ds(off, 128)]],
            buf.at[pl.ds(0, 128)], sem)
        c2 = pltpu.make_async_copy(
            table_hbm.at[idx_all.at[pl.ds(off + 128, L - 128)]],
            buf.at[pl.ds(128, L - 128)], sem)
        return c1, c2

    def row_step(r, b):
        buf, sem = bufs[b], sems[b]
        off = r * L
        n0v = jnp.zeros((LANES,), jnp.int32)
        for k in range(NFULL):
            v = idx_all[pl.ds(off + k * LANES, LANES)]
            n0v = n0v + plsc.all_reduce_population_count(v == 0)
        v = idx_all[pl.ds(off + NFULL * LANES, LANES)]
        n0v = n0v + plsc.all_reduce_population_count((v == 0) & last_mask)

        c1, c2 = gather_copies(r, buf, sem)
        c1.wait()
        c2.wait()

        def acc_body(j, accs):
            return tuple(accs[k] + buf[j, pl.ds(k * LANES, LANES)]
                         for k in range(NVR))
        accs = lax.fori_loop(0, L, acc_body, (zero, zero, zero, zero),
                             unroll=8)

        rn = jnp.minimum(r + NBUF, RPW - 1)
        f1, f2 = gather_copies(rn, buf, sem)
        f1.start()
        f2.start()

        n0f = n0v.astype(jnp.float32)
        cntv = jnp.float32(L) - n0f
        scalev = jnp.where(cntv > 0.0, 1.0 / jnp.maximum(cntv, 1.0), 0.0)
        for k in range(NVR):
            out_blk[pl.ds(r * D + k * LANES, LANES)] = \
                (accs[k] - n0f * t0[k]) * scalev

    for b in range(NBUF):
        c1, c2 = gather_copies(jnp.int32(b), bufs[b], sems[b])
        c1.start()
        c2.start()

    def body(i, carry):
        for b in range(NBUF):
            row_step(i * NBUF + b, b)
        return carry

    lax.fori_loop(0, RPW // NBUF, body, 0)

    for b in range(NBUF):
        c1, c2 = gather_copies(0, bufs[b], sems[b])
        c1.wait()
        c2.wait()

    pltpu.sync_copy(out_blk, out_hbm.at[pl.ds(row0 * D, RPW * D)])


_sc_call = functools.partial(
    pl.kernel,
    mesh=plsc.VectorSubcoreMesh(core_axis_name="c", subcore_axis_name="s"),
    out_type=jax.ShapeDtypeStruct((B * D,), jnp.float32),
    compiler_params=pltpu.CompilerParams(
        needs_layout_passes=False, use_tc_tiling_on_sc=False),
    scratch_types=[
        pltpu.VMEM((RPW * L + LANES,), jnp.int32),
        pltpu.VMEM((L, D), jnp.float32),
        pltpu.VMEM((L, D), jnp.float32),
        pltpu.VMEM((L, D), jnp.float32),
        pltpu.VMEM((L, D), jnp.float32),
        pltpu.VMEM((D,), jnp.float32),
        pltpu.VMEM((RPW * D,), jnp.float32),
        pltpu.SemaphoreType.DMA,
        pltpu.SemaphoreType.DMA,
        pltpu.SemaphoreType.DMA,
        pltpu.SemaphoreType.DMA,
    ],
)(_tec_body)


V = 1000000
NTFULL = V // 128
VREM = V - NTFULL * 128
GRP = 4
SLOTS = 248

_lane = None


def _transpose_body(t_hbm, rag_hbm, p_hbm, *refs):
    in_bufs = refs[0:GRP]
    out_bufs = refs[GRP:2 * GRP]
    in_sems = refs[2 * GRP:3 * GRP]
    out_sems = refs[3 * GRP:4 * GRP]

    wid = lax.axis_index("s") * _NC + lax.axis_index("c")
    n_w = (NTFULL - 1 - wid) // NW + 1
    lane = lax.iota(jnp.int32, LANES)
    rvs = [(c * LANES + lane) // 2 for c in range(8)]
    cbs = [(c * LANES + lane) % 2 * D for c in range(8)]

    def slot_j(t):
        return wid + NW * jnp.minimum(t, n_w - 1)

    def in_copy(t, b):
        j = slot_j(t)
        return pltpu.make_async_copy(
            t_hbm.at[:, pl.ds(j * 128, 128)], in_bufs[b], in_sems[b])

    def out_copy(t, b):
        j = slot_j(t)
        return pltpu.make_async_copy(
            out_bufs[b].at[:, pl.ds(0, 128)],
            p_hbm.at[pl.ds(j * D, D)], out_sems[b])

    def transpose_block(src, dst, ncols):
        def dbody(d, carry):
            for c in range(ncols // LANES):
                x = src[d, pl.ds(c * LANES, LANES)]
                plsc.store_scatter(dst, [rvs[c], cbs[c] + d], x)
            return carry
        lax.fori_loop(0, D, dbody, 0, unroll=4)

    for b in range(GRP):
        in_copy(jnp.int32(b), b).start()

    def group_body(q, carry):
        base = q * GRP
        for b in range(GRP):
            t = base + b
            in_copy(t, b).wait()
            transpose_block(in_bufs[b], out_bufs[b], 128)
            in_copy(jnp.minimum(t + GRP, SLOTS - 1), b).start()
            out_copy(t, b).start()
        for b in range(GRP):
            out_copy(base + b, b).wait()
        return carry

    lax.fori_loop(0, SLOTS // GRP, group_body, 0)

    for b in range(GRP):
        in_copy(0, b).wait()

    @pl.when(wid == (NTFULL % NW))
    def _():
        pltpu.sync_copy(rag_hbm, in_bufs[0].at[pl.ds(0, VREM // 2)])
        pltpu.sync_copy(in_bufs[0].at[pl.ds(0, VREM // 2)],
                        p_hbm.at[pl.ds(NTFULL * D, VREM // 2)])


_transpose_call = functools.partial(
    pl.kernel,
    mesh=plsc.VectorSubcoreMesh(core_axis_name="c", subcore_axis_name="s"),
    out_type=jax.ShapeDtypeStruct((V // 2, 2 * D), jnp.float32),
    compiler_params=pltpu.CompilerParams(
        needs_layout_passes=False, use_tc_tiling_on_sc=True),
    scratch_types=(
        [pltpu.VMEM((D, 128), jnp.float32) for _ in range(GRP)] +
        [pltpu.VMEM((D, 129), jnp.float32) for _ in range(GRP)] +
        [pltpu.SemaphoreType.DMA for _ in range(2 * GRP)]
    ),
)(_transpose_body)


def kernel(inputs, table):
    idx_flat = inputs.reshape(-1).astype(jnp.int32)
    rag = table[NTFULL * 128:].reshape(VREM // 2, 2 * D)
    packed = _transpose_call(table.T, rag)
    table_lin = packed.reshape(V, D)
    out_flat = _sc_call(idx_flat, table_lin)
    return out_flat.reshape(B, 1, D)


# speedup vs baseline: 1.3612x; 1.3612x over previous
import functools

import jax
import jax.numpy as jnp
from jax import lax
from jax.experimental import pallas as pl
from jax.experimental.pallas import tpu as pltpu
from jax.experimental.pallas import tpu_sc as plsc

B = 4096
L = 200
D = 64
LANES = 16
NVR = D // LANES
NFULL = L // LANES
LREM = L - NFULL * LANES
NBUF = 4

_info = plsc.get_sparse_core_info()
_NC, _NS = _info.num_cores, _info.num_subcores
NW = _NC * _NS
RPW = B // NW


def _tec_body(idx_hbm, table_hbm, out_hbm, idx_all,
              rows0, rows1, rows2, rows3, t0_v, out_blk,
              sem0, sem1, sem2, sem3):
    bufs = (rows0, rows1, rows2, rows3)
    sems = (sem0, sem1, sem2, sem3)
    wid = lax.axis_index("s") * _NC + lax.axis_index("c")
    row0 = wid * RPW

    pltpu.sync_copy(idx_hbm.at[pl.ds(row0 * L, RPW * L)],
                    idx_all.at[pl.ds(0, RPW * L)])
    pltpu.sync_copy(table_hbm.at[0], t0_v)

    lane = lax.iota(jnp.int32, LANES)
    last_mask = lane < LREM
    zero = jnp.zeros((LANES,), jnp.float32)
    t0 = [t0_v[pl.ds(k * LANES, LANES)] for k in range(NVR)]

    def gather_copies(r, buf, sem):
        off = r * L
        c1 = pltpu.make_async_copy(
            table_hbm.at[idx_all.at[pl.ds(off, 128)]],
            buf.at[pl.ds(0, 128)], sem)
        c2 = pltpu.make_async_copy(
            table_hbm.at[idx_all.at[pl.ds(off + 128, L - 128)]],
            buf.at[pl.ds(128, L - 128)], sem)
        return c1, c2

    def row_step(r, b):
        buf, sem = bufs[b], sems[b]
        off = r * L
        n0v = jnp.zeros((LANES,), jnp.int32)
        for k in range(NFULL):
            v = idx_all[pl.ds(off + k * LANES, LANES)]
            n0v = n0v + plsc.all_reduce_population_count(v == 0)
        v = idx_all[pl.ds(off + NFULL * LANES, LANES)]
        n0v = n0v + plsc.all_reduce_population_count((v == 0) & last_mask)

        c1, c2 = gather_copies(r, buf, sem)
        c1.wait()
        c2.wait()

        def acc_body(j, accs):
            return tuple(accs[k] + buf[j, pl.ds(k * LANES, LANES)]
                         for k in range(NVR))
        accs = lax.fori_loop(0, L, acc_body, (zero, zero, zero, zero),
                             unroll=8)

        rn = jnp.minimum(r + NBUF, RPW - 1)
        f1, f2 = gather_copies(rn, buf, sem)
        f1.start()
        f2.start()

        n0f = n0v.astype(jnp.float32)
        cntv = jnp.float32(L) - n0f
        scalev = jnp.where(cntv > 0.0, 1.0 / jnp.maximum(cntv, 1.0), 0.0)
        for k in range(NVR):
            out_blk[pl.ds(r * D + k * LANES, LANES)] = \
                (accs[k] - n0f * t0[k]) * scalev

    for b in range(NBUF):
        c1, c2 = gather_copies(jnp.int32(b), bufs[b], sems[b])
        c1.start()
        c2.start()

    def body(i, carry):
        for b in range(NBUF):
            row_step(i * NBUF + b, b)
        return carry

    lax.fori_loop(0, RPW // NBUF, body, 0)

    for b in range(NBUF):
        c1, c2 = gather_copies(0, bufs[b], sems[b])
        c1.wait()
        c2.wait()

    pltpu.sync_copy(out_blk, out_hbm.at[pl.ds(row0 * D, RPW * D)])


_sc_call = functools.partial(
    pl.kernel,
    mesh=plsc.VectorSubcoreMesh(core_axis_name="c", subcore_axis_name="s"),
    out_type=jax.ShapeDtypeStruct((B * D,), jnp.float32),
    compiler_params=pltpu.CompilerParams(
        needs_layout_passes=False, use_tc_tiling_on_sc=False),
    scratch_types=[
        pltpu.VMEM((RPW * L + LANES,), jnp.int32),
        pltpu.VMEM((L, D), jnp.float32),
        pltpu.VMEM((L, D), jnp.float32),
        pltpu.VMEM((L, D), jnp.float32),
        pltpu.VMEM((L, D), jnp.float32),
        pltpu.VMEM((D,), jnp.float32),
        pltpu.VMEM((RPW * D,), jnp.float32),
        pltpu.SemaphoreType.DMA,
        pltpu.SemaphoreType.DMA,
        pltpu.SemaphoreType.DMA,
        pltpu.SemaphoreType.DMA,
    ],
)(_tec_body)


V = 1000000
NTFULL = V // 128
VREM = V - NTFULL * 128
GRP = 4
SLOTS = 248

_lane = None


def _transpose_body(t_hbm, rag_hbm, p_hbm, *refs):
    in_bufs = refs[0:GRP]
    out_bufs = refs[GRP:2 * GRP]
    in_sems = refs[2 * GRP:3 * GRP]
    out_sems = refs[3 * GRP:4 * GRP]

    wid = lax.axis_index("s") * _NC + lax.axis_index("c")
    n_w = (NTFULL - 1 - wid) // NW + 1
    lane = lax.iota(jnp.int32, LANES)
    rvs = [(c * LANES + lane) // 2 for c in range(8)]
    cbs = [(c * LANES + lane) % 2 * D for c in range(8)]

    def slot_j(t):
        return wid + NW * jnp.minimum(t, n_w - 1)

    def in_copy(t, b):
        j = slot_j(t)
        return pltpu.make_async_copy(
            t_hbm.at[:, pl.ds(j * 128, 128)], in_bufs[b], in_sems[b])

    def out_copy(t, b):
        j = slot_j(t)
        return pltpu.make_async_copy(
            out_bufs[b].at[:, pl.ds(0, 128)],
            p_hbm.at[pl.ds(j * D, D)], out_sems[b])

    def transpose_block(src, dst, ncols):
        @plsc.parallel_loop(0, D, unroll=4)
        def _(d):
            for c in range(ncols // LANES):
                x = src[d, pl.ds(c * LANES, LANES)]
                plsc.store_scatter(dst, [rvs[c], cbs[c] + d], x)

    for b in range(GRP):
        in_copy(jnp.int32(b), b).start()

    def group_body(q, carry):
        base = q * GRP
        for b in range(GRP):
            t = base + b
            in_copy(t, b).wait()
            transpose_block(in_bufs[b], out_bufs[b], 128)
            in_copy(jnp.minimum(t + GRP, SLOTS - 1), b).start()
            out_copy(t, b).start()
        for b in range(GRP):
            out_copy(base + b, b).wait()
        return carry

    lax.fori_loop(0, SLOTS // GRP, group_body, 0)

    for b in range(GRP):
        in_copy(0, b).wait()

    @pl.when(wid == (NTFULL % NW))
    def _():
        pltpu.sync_copy(rag_hbm, in_bufs[0].at[pl.ds(0, VREM // 2)])
        pltpu.sync_copy(in_bufs[0].at[pl.ds(0, VREM // 2)],
                        p_hbm.at[pl.ds(NTFULL * D, VREM // 2)])


_transpose_call = functools.partial(
    pl.kernel,
    mesh=plsc.VectorSubcoreMesh(core_axis_name="c", subcore_axis_name="s"),
    out_type=jax.ShapeDtypeStruct((V // 2, 2 * D), jnp.float32),
    compiler_params=pltpu.CompilerParams(
        needs_layout_passes=False, use_tc_tiling_on_sc=True),
    scratch_types=(
        [pltpu.VMEM((D, 128), jnp.float32) for _ in range(GRP)] +
        [pltpu.VMEM((D, 129), jnp.float32) for _ in range(GRP)] +
        [pltpu.SemaphoreType.DMA for _ in range(2 * GRP)]
    ),
)(_transpose_body)


def kernel(inputs, table):
    idx_flat = inputs.reshape(-1).astype(jnp.int32)
    rag = table[NTFULL * 128:].reshape(VREM // 2, 2 * D)
    packed = _transpose_call(table.T, rag)
    table_lin = packed.reshape(V, D)
    out_flat = _sc_call(idx_flat, table_lin)
    return out_flat.reshape(B, 1, D)
